# baseline (speedup 1.0000x reference)
import jax
import jax.numpy as jnp
from jax import lax
from jax.experimental import pallas as pl
from jax.experimental.pallas import tpu as pltpu
from jax.experimental.pallas import tpu_sc as plsc

B = 16384
K = 64
NW = 32
BPW = B // NW
NCH = 4
CH = BPW // NCH
L = 16
NG = BPW // L


def _body(x_hbm, ut_hbm, it_hbm, wbt_hbm, out_hbm,
          x_v, uidx_v, iidx_v, urows_v, irows_v, wbt_v, acc_v, out_v,
          sem_u, sem_i):
    c = lax.axis_index("c")
    s = lax.axis_index("s")
    wid = s * 2 + c
    base = wid * BPW

    pltpu.sync_copy(wbt_hbm, wbt_v)
    pltpu.sync_copy(x_hbm.at[pl.ds(2 * base, 2 * BPW)], x_v)

    def deint(g, carry):
        jl2 = 2 * (g * L + lax.iota(jnp.int32, L))
        u = plsc.load_gather(x_v, [jl2])
        i = plsc.load_gather(x_v, [jl2 + 1])
        ch = g // (CH // L)
        off = (g % (CH // L)) * L
        uidx_v[ch, pl.ds(off, L)] = u
        iidx_v[ch, pl.ds(off, L)] = i
        return carry

    lax.fori_loop(0, BPW // L, deint, 0)

    copies = []
    for ci in range(NCH):
        copies.append(pltpu.async_copy(
            ut_hbm.at[uidx_v.at[ci]], urows_v.at[pl.ds(ci * CH, CH), :], sem_u))
        copies.append(pltpu.async_copy(
            it_hbm.at[iidx_v.at[ci]], irows_v.at[pl.ds(ci * CH, CH), :], sem_i))
    for cp in copies:
        cp.wait()

    bias_vec = wbt_v[2 * K, :]
    for q in range(NG):
        acc_v[pl.ds(q * L, L)] = bias_vec

    iota = lax.iota(jnp.int32, L)
    step = jnp.full((L,), L, jnp.int32)

    def kbody(k, carry):
        wu = wbt_v[k, :]
        wi = wbt_v[k + K, :]
        kv = jnp.full((L,), k, jnp.int32)
        jl = iota
        for q in range(NG):
            cu = plsc.load_gather(urows_v, [jl, kv])
            ci2 = plsc.load_gather(irows_v, [jl, kv])
            plsc.addupdate(acc_v.at[pl.ds(q * L, L)], cu * wu + ci2 * wi)
            jl = jl + step
        return carry

    lax.fori_loop(0, K, kbody, 0)

    for q in range(NG):
        z = acc_v[pl.ds(q * L, L)]
        out_v[pl.ds(q * L, L)] = 1.0 / (1.0 + jnp.exp(-z))
    pltpu.sync_copy(out_v, out_hbm.at[pl.ds(base, BPW)])


_mesh = plsc.VectorSubcoreMesh(
    core_axis_name="c", subcore_axis_name="s", num_cores=2, num_subcores=16)

_sc_call = pl.kernel(
    _body,
    out_type=jax.ShapeDtypeStruct((B,), jnp.float32),
    mesh=_mesh,
    compiler_params=pltpu.CompilerParams(
        needs_layout_passes=False, use_tc_tiling_on_sc=False),
    scratch_types=[
        pltpu.VMEM((2 * BPW,), jnp.int32),
        pltpu.VMEM((NCH, CH), jnp.int32),
        pltpu.VMEM((NCH, CH), jnp.int32),
        pltpu.VMEM((BPW, K), jnp.float32),
        pltpu.VMEM((BPW, K), jnp.float32),
        pltpu.VMEM((2 * K + 1, L), jnp.float32),
        pltpu.VMEM((BPW,), jnp.float32),
        pltpu.VMEM((BPW,), jnp.float32),
        pltpu.SemaphoreType.DMA,
        pltpu.SemaphoreType.DMA,
    ],
)


@jax.jit
def kernel(x, user_table, item_table, W, b):
    wb = jnp.concatenate([W.reshape(-1), b.reshape(-1)])
    wbt = jnp.tile(wb[:, None], (1, L))
    return _sc_call(x.reshape(-1), user_table, item_table, wbt)


# speedup vs baseline: 4.0201x; 4.0201x over previous
import jax
import jax.numpy as jnp
from jax import lax
from jax.experimental import pallas as pl
from jax.experimental.pallas import tpu as pltpu
from jax.experimental.pallas import tpu_sc as plsc

B = 16384
K = 64
N = 1000000
BN = 4096
NW = 32
BPW = B // NW
NCH = 4
CH = BPW // NCH
L = 16


def _scores_body(ut_ref, it_ref, wu_ref, wi_ref, su_ref, si_ref):
    su_ref[...] = jnp.sum(ut_ref[...] * wu_ref[...], axis=0)
    si_ref[...] = jnp.sum(it_ref[...] * wi_ref[...], axis=0)


_scores_call = pl.pallas_call(
    _scores_body,
    grid=(pl.cdiv(N, BN),),
    in_specs=[
        pl.BlockSpec((K, BN), lambda n: (0, n)),
        pl.BlockSpec((K, BN), lambda n: (0, n)),
        pl.BlockSpec((K, 1), lambda n: (0, 0)),
        pl.BlockSpec((K, 1), lambda n: (0, 0)),
    ],
    out_specs=[
        pl.BlockSpec((BN,), lambda n: (n,)),
        pl.BlockSpec((BN,), lambda n: (n,)),
    ],
    out_shape=[
        jax.ShapeDtypeStruct((N,), jnp.float32),
        jax.ShapeDtypeStruct((N,), jnp.float32),
    ],
)


def _gather_body(x_hbm, su_hbm, si_hbm, bias_hbm, out_hbm,
                 x_v, uidx_v, iidx_v, sv_v, bias_v, out_v, sem_u, sem_i):
    c = lax.axis_index("c")
    s = lax.axis_index("s")
    wid = s * 2 + c
    base = wid * BPW

    pltpu.sync_copy(bias_hbm, bias_v)
    pltpu.sync_copy(x_hbm.at[pl.ds(2 * base, 2 * BPW)], x_v)

    def deint(g, carry):
        jl2 = 2 * (g * L + lax.iota(jnp.int32, L))
        u = plsc.load_gather(x_v, [jl2])
        i = plsc.load_gather(x_v, [jl2 + 1])
        ch = g // (CH // L)
        off = (g % (CH // L)) * L
        uidx_v[ch, pl.ds(off, L)] = u
        iidx_v[ch, pl.ds(off, L)] = i
        return carry

    lax.fori_loop(0, BPW // L, deint, 0)

    copies = []
    for ci in range(NCH):
        copies.append(pltpu.async_copy(
            su_hbm.at[uidx_v.at[ci]],
            sv_v.at[pl.ds(ci * CH, CH)], sem_u))
        copies.append(pltpu.async_copy(
            si_hbm.at[iidx_v.at[ci]],
            sv_v.at[pl.ds(BPW + ci * CH, CH)], sem_i))
    for cp in copies:
        cp.wait()

    bias = bias_v[pl.ds(0, L)]
    for q in range(BPW // L):
        z = sv_v[pl.ds(q * L, L)] + sv_v[pl.ds(BPW + q * L, L)] + bias
        out_v[pl.ds(q * L, L)] = 1.0 / (1.0 + jnp.exp(-z))
    pltpu.sync_copy(out_v, out_hbm.at[pl.ds(base, BPW)])


_mesh = plsc.VectorSubcoreMesh(
    core_axis_name="c", subcore_axis_name="s", num_cores=2, num_subcores=16)

_gather_call = pl.kernel(
    _gather_body,
    out_type=jax.ShapeDtypeStruct((B,), jnp.float32),
    mesh=_mesh,
    compiler_params=pltpu.CompilerParams(
        needs_layout_passes=False, use_tc_tiling_on_sc=False),
    scratch_types=[
        pltpu.VMEM((2 * BPW,), jnp.int32),
        pltpu.VMEM((NCH, CH), jnp.int32),
        pltpu.VMEM((NCH, CH), jnp.int32),
        pltpu.VMEM((2 * BPW,), jnp.float32),
        pltpu.VMEM((L,), jnp.float32),
        pltpu.VMEM((BPW,), jnp.float32),
        pltpu.SemaphoreType.DMA,
        pltpu.SemaphoreType.DMA,
    ],
)


@jax.jit
def kernel(x, user_table, item_table, W, b):
    wu = W[:K]
    wi = W[K:]
    su, si = _scores_call(user_table.T, item_table.T, wu, wi)
    bias_t = jnp.tile(b, (L,))
    return _gather_call(x.reshape(-1), su, si, bias_t)


# speedup vs baseline: 5.2370x; 1.3027x over previous
import jax
import jax.numpy as jnp
from jax import lax
from jax.experimental import pallas as pl
from jax.experimental.pallas import tpu as pltpu
from jax.experimental.pallas import tpu_sc as plsc

B = 16384
K = 64
N = 1000000
BN = 8192
NW = 32
BPW = B // NW
NCH = 4
CH = BPW // NCH
L = 16


def _scores_body(ut_ref, it_ref, wu_ref, wi_ref, su_ref, si_ref):
    su_ref[...] = jnp.sum(ut_ref[...] * wu_ref[...], axis=0)
    si_ref[...] = jnp.sum(it_ref[...] * wi_ref[...], axis=0)


_scores_call = pl.pallas_call(
    _scores_body,
    grid=(pl.cdiv(N, BN),),
    in_specs=[
        pl.BlockSpec((K, BN), lambda n: (0, n)),
        pl.BlockSpec((K, BN), lambda n: (0, n)),
        pl.BlockSpec((K, 1), lambda n: (0, 0)),
        pl.BlockSpec((K, 1), lambda n: (0, 0)),
    ],
    out_specs=[
        pl.BlockSpec((BN,), lambda n: (n,)),
        pl.BlockSpec((BN,), lambda n: (n,)),
    ],
    out_shape=[
        jax.ShapeDtypeStruct((N,), jnp.float32),
        jax.ShapeDtypeStruct((N,), jnp.float32),
    ],
)


def _gather_body(x_hbm, su_hbm, si_hbm, bias_hbm, out_hbm,
                 x_v, uidx_v, iidx_v, sv_v, bias_v, out_v, sem_u, sem_i):
    c = lax.axis_index("c")
    s = lax.axis_index("s")
    wid = s * 2 + c
    base = wid * BPW

    pltpu.sync_copy(bias_hbm, bias_v)
    pltpu.sync_copy(x_hbm.at[pl.ds(2 * base, 2 * BPW)], x_v)

    def deint(g, carry):
        jl2 = 2 * (g * L + lax.iota(jnp.int32, L))
        u = plsc.load_gather(x_v, [jl2])
        i = plsc.load_gather(x_v, [jl2 + 1])
        ch = g // (CH // L)
        off = (g % (CH // L)) * L
        uidx_v[ch, pl.ds(off, L)] = u
        iidx_v[ch, pl.ds(off, L)] = i
        return carry

    lax.fori_loop(0, BPW // L, deint, 0)

    copies = []
    for ci in range(NCH):
        copies.append(pltpu.async_copy(
            su_hbm.at[uidx_v.at[ci]],
            sv_v.at[pl.ds(ci * CH, CH)], sem_u))
        copies.append(pltpu.async_copy(
            si_hbm.at[iidx_v.at[ci]],
            sv_v.at[pl.ds(BPW + ci * CH, CH)], sem_i))
    for cp in copies:
        cp.wait()

    bias = bias_v[pl.ds(0, L)]
    for q in range(BPW // L):
        z = sv_v[pl.ds(q * L, L)] + sv_v[pl.ds(BPW + q * L, L)] + bias
        out_v[pl.ds(q * L, L)] = 1.0 / (1.0 + jnp.exp(-z))
    pltpu.sync_copy(out_v, out_hbm.at[pl.ds(base, BPW)])


_mesh = plsc.VectorSubcoreMesh(
    core_axis_name="c", subcore_axis_name="s", num_cores=2, num_subcores=16)

_gather_call = pl.kernel(
    _gather_body,
    out_type=jax.ShapeDtypeStruct((B,), jnp.float32),
    mesh=_mesh,
    compiler_params=pltpu.CompilerParams(
        needs_layout_passes=False, use_tc_tiling_on_sc=False),
    scratch_types=[
        pltpu.VMEM((2 * BPW,), jnp.int32),
        pltpu.VMEM((NCH, CH), jnp.int32),
        pltpu.VMEM((NCH, CH), jnp.int32),
        pltpu.VMEM((2 * BPW,), jnp.float32),
        pltpu.VMEM((L,), jnp.float32),
        pltpu.VMEM((BPW,), jnp.float32),
        pltpu.SemaphoreType.DMA,
        pltpu.SemaphoreType.DMA,
    ],
)


@jax.jit
def kernel(x, user_table, item_table, W, b):
    wu = W[:K]
    wi = W[K:]
    su, si = _scores_call(user_table.T, item_table.T, wu, wi)
    bias_t = jnp.tile(b, (L,))
    return _gather_call(x.reshape(-1), su, si, bias_t)


# speedup vs baseline: 6.1030x; 1.1654x over previous
import jax
import jax.numpy as jnp
from jax import lax
from jax.experimental import pallas as pl
from jax.experimental.pallas import tpu as pltpu
from jax.experimental.pallas import tpu_sc as plsc

B = 16384
K = 64
N = 1000000
BN = 16384
NW = 32
BPW = B // NW
NCH = 4
CH = BPW // NCH
L = 16


def _scores_body(ut_ref, it_ref, wu_ref, wi_ref, su_ref, si_ref):
    su_ref[...] = jnp.sum(ut_ref[...] * wu_ref[...], axis=0)
    si_ref[...] = jnp.sum(it_ref[...] * wi_ref[...], axis=0)


_scores_call = pl.pallas_call(
    _scores_body,
    grid=(pl.cdiv(N, BN),),
    in_specs=[
        pl.BlockSpec((K, BN), lambda n: (0, n)),
        pl.BlockSpec((K, BN), lambda n: (0, n)),
        pl.BlockSpec((K, 1), lambda n: (0, 0)),
        pl.BlockSpec((K, 1), lambda n: (0, 0)),
    ],
    out_specs=[
        pl.BlockSpec((BN,), lambda n: (n,)),
        pl.BlockSpec((BN,), lambda n: (n,)),
    ],
    out_shape=[
        jax.ShapeDtypeStruct((N,), jnp.float32),
        jax.ShapeDtypeStruct((N,), jnp.float32),
    ],
)


def _gather_body(x_hbm, su_hbm, si_hbm, bias_hbm, out_hbm,
                 x_v, uidx_v, iidx_v, sv_v, bias_v, out_v, sem_u, sem_i):
    c = lax.axis_index("c")
    s = lax.axis_index("s")
    wid = s * 2 + c
    base = wid * BPW

    pltpu.sync_copy(bias_hbm, bias_v)
    pltpu.sync_copy(x_hbm.at[pl.ds(2 * base, 2 * BPW)], x_v)

    def deint(g, carry):
        jl2 = 2 * (g * L + lax.iota(jnp.int32, L))
        u = plsc.load_gather(x_v, [jl2])
        i = plsc.load_gather(x_v, [jl2 + 1])
        ch = g // (CH // L)
        off = (g % (CH // L)) * L
        uidx_v[ch, pl.ds(off, L)] = u
        iidx_v[ch, pl.ds(off, L)] = i
        return carry

    lax.fori_loop(0, BPW // L, deint, 0)

    copies = []
    for ci in range(NCH):
        copies.append(pltpu.async_copy(
            su_hbm.at[uidx_v.at[ci]],
            sv_v.at[pl.ds(ci * CH, CH)], sem_u))
        copies.append(pltpu.async_copy(
            si_hbm.at[iidx_v.at[ci]],
            sv_v.at[pl.ds(BPW + ci * CH, CH)], sem_i))
    for cp in copies:
        cp.wait()

    bias = bias_v[pl.ds(0, L)]
    for q in range(BPW // L):
        z = sv_v[pl.ds(q * L, L)] + sv_v[pl.ds(BPW + q * L, L)] + bias
        out_v[pl.ds(q * L, L)] = 1.0 / (1.0 + jnp.exp(-z))
    pltpu.sync_copy(out_v, out_hbm.at[pl.ds(base, BPW)])


_mesh = plsc.VectorSubcoreMesh(
    core_axis_name="c", subcore_axis_name="s", num_cores=2, num_subcores=16)

_gather_call = pl.kernel(
    _gather_body,
    out_type=jax.ShapeDtypeStruct((B,), jnp.float32),
    mesh=_mesh,
    compiler_params=pltpu.CompilerParams(
        needs_layout_passes=False, use_tc_tiling_on_sc=False),
    scratch_types=[
        pltpu.VMEM((2 * BPW,), jnp.int32),
        pltpu.VMEM((NCH, CH), jnp.int32),
        pltpu.VMEM((NCH, CH), jnp.int32),
        pltpu.VMEM((2 * BPW,), jnp.float32),
        pltpu.VMEM((L,), jnp.float32),
        pltpu.VMEM((BPW,), jnp.float32),
        pltpu.SemaphoreType.DMA,
        pltpu.SemaphoreType.DMA,
    ],
)


@jax.jit
def kernel(x, user_table, item_table, W, b):
    wu = W[:K]
    wi = W[K:]
    su, si = _scores_call(user_table.T, item_table.T, wu, wi)
    bias_t = jnp.tile(b, (L,))
    return _gather_call(x.reshape(-1), su, si, bias_t)
